# baseline (speedup 1.0000x reference)
import functools

import jax
import jax.numpy as jnp
from jax import lax
from jax.experimental import pallas as pl
from jax.experimental.pallas import tpu as pltpu
from jax.experimental.pallas import tpu_sc as plsc

B, C, V_LVL, V_PREV, PATCH = 8, 128, 40962, 10242, 7
ROWS = B * C
NW = 32
ROWS_PER_W = ROWS // NW
P_PAD = 10256
N_CHUNKS = P_PAD // 16
OUT_ROW = 10248
XROW_PAD = 40968
COMB_LEN = PATCH * P_PAD


@functools.partial(
    pl.kernel,
    out_type=jax.ShapeDtypeStruct((ROWS * OUT_ROW,), jnp.float32),
    mesh=plsc.VectorSubcoreMesh(core_axis_name="c", subcore_axis_name="s"),
    compiler_params=pltpu.CompilerParams(needs_layout_passes=False),
    scratch_types=[
        pltpu.VMEM((COMB_LEN,), jnp.int32),
        pltpu.VMEM((XROW_PAD,), jnp.float32),
        pltpu.VMEM((P_PAD,), jnp.float32),
    ],
)
def _sc_maxpool(x_hbm, v2p_hbm, patches_hbm, out_hbm, comb_v, xrow_v, orow_v):
    wid = lax.axis_index("s") * 2 + lax.axis_index("c")

    pltpu.sync_copy(v2p_hbm, xrow_v)
    pltpu.sync_copy(patches_hbm, comb_v)

    def compose(t, carry):
        off = t * 16
        pidx = comb_v[pl.ds(off, 16)]
        vals = plsc.load_gather(xrow_v, [pidx])
        comb_v[pl.ds(off, 16)] = plsc.bitcast(vals, jnp.int32)
        return carry

    lax.fori_loop(0, COMB_LEN // 16, compose, 0)

    def do_row(i, carry):
        r = wid * ROWS_PER_W + i
        base = r * V_LVL
        delta = lax.bitwise_and(base, 7)
        a = pl.multiple_of(base - delta, 8)
        pltpu.sync_copy(x_hbm.at[pl.ds(a, XROW_PAD)], xrow_v)

        def chunk(ci, carry2):
            off = ci * 16
            idx = comb_v[pl.ds(off, 16)] + delta
            m = plsc.load_gather(xrow_v, [idx])
            for j in range(1, PATCH):
                idx = comb_v[pl.ds(j * P_PAD + off, 16)] + delta
                m = jnp.maximum(m, plsc.load_gather(xrow_v, [idx]))
            orow_v[pl.ds(off, 16)] = m
            return carry2

        lax.fori_loop(0, N_CHUNKS, chunk, 0)
        pltpu.sync_copy(orow_v.at[pl.ds(0, OUT_ROW)],
                        out_hbm.at[pl.ds(r * OUT_ROW, OUT_ROW)])
        return carry

    lax.fori_loop(0, ROWS_PER_W, do_row, 0)


def kernel(x, vertices_to_prev_lvl, neihboring_patches):
    x_flat = x.reshape(-1)
    v2p_f = lax.bitcast_convert_type(
        jnp.pad(vertices_to_prev_lvl, (0, XROW_PAD - V_LVL)), jnp.float32)
    patches_flat = jnp.pad(
        neihboring_patches.T, ((0, 0), (0, P_PAD - V_PREV))).reshape(-1)
    out_flat = _sc_maxpool(x_flat, v2p_f, patches_flat)
    return out_flat.reshape(ROWS, OUT_ROW)[:, :V_PREV].reshape(B, C, V_PREV)
